# baseline (device time: 674565 ns/iter reference)
import jax
import jax.numpy as jnp
from jax import lax
from jax.experimental import pallas as pl
from jax.experimental.pallas import tpu as pltpu

N_DEV = 32
S = 1024
DM = 2048
DKV = 4096
H, DH, DR = 16, 128, 32
ROWS = S // N_DEV

_sem_signal = getattr(pl, "semaphore_signal", None) or pltpu.semaphore_signal
_sem_wait = getattr(pl, "semaphore_wait", None) or pltpu.semaphore_wait
_DevIdTy = getattr(pl, "DeviceIdType", None) or pltpu.DeviceIdType
_CompilerParams = getattr(pltpu, "CompilerParams", None) or pltpu.TPUCompilerParams


def _neighbor_barrier(left, right):
    barrier = pltpu.get_barrier_semaphore()
    for nbr in (left, right):
        _sem_signal(barrier, inc=1, device_id=(nbr,), device_id_type=_DevIdTy.MESH)
    _sem_wait(barrier, 2)


def _allreduce_body(kvp_ref, out_ref, stage_ref, send_sems, recv_sems, credit_sems):
    d = lax.axis_index("i")
    left = jnp.mod(d - 1, N_DEV)
    right = jnp.mod(d + 1, N_DEV)
    _neighbor_barrier(left, right)

    out_ref[...] = kvp_ref[...]

    n_rs = N_DEV - 1
    n_steps = 2 * (N_DEV - 1)

    for u in range(n_steps):
        slot = u % 2
        if u >= 2:
            _sem_wait(credit_sems.at[slot], 1)

        if u < n_rs:
            s = u
            send_chunk = jnp.mod(d - s, N_DEV)
            recv_chunk = jnp.mod(d - s - 1, N_DEV)
            rdma = pltpu.make_async_remote_copy(
                src_ref=out_ref.at[pl.ds(send_chunk * ROWS, ROWS), :],
                dst_ref=stage_ref.at[slot],
                send_sem=send_sems.at[slot],
                recv_sem=recv_sems.at[slot],
                device_id=(right,),
                device_id_type=_DevIdTy.MESH,
            )
            rdma.start()
            rdma.wait()
            rr = pl.ds(recv_chunk * ROWS, ROWS)
            out_ref[rr, :] = out_ref[rr, :] + stage_ref[slot]
        else:
            t = u - n_rs
            send_chunk = jnp.mod(d + 1 - t, N_DEV)
            recv_chunk = jnp.mod(d - t, N_DEV)
            sr = pl.ds(send_chunk * ROWS, ROWS)
            rr = pl.ds(recv_chunk * ROWS, ROWS)
            send_desc = pltpu.make_async_remote_copy(
                src_ref=out_ref.at[sr, :],
                dst_ref=out_ref.at[sr, :],
                send_sem=send_sems.at[slot],
                recv_sem=recv_sems.at[slot],
                device_id=(right,),
                device_id_type=_DevIdTy.MESH,
            )
            send_desc.start()
            recv_desc = pltpu.make_async_remote_copy(
                src_ref=out_ref.at[rr, :],
                dst_ref=out_ref.at[rr, :],
                send_sem=send_sems.at[slot],
                recv_sem=recv_sems.at[slot],
                device_id=(right,),
                device_id_type=_DevIdTy.MESH,
            )
            send_desc.wait_send()
            recv_desc.wait_recv()

        if u <= n_steps - 3:
            _sem_signal(
                credit_sems.at[slot], inc=1, device_id=(left,),
                device_id_type=_DevIdTy.MESH,
            )


def _pallas_allreduce(kvp):
    return pl.pallas_call(
        _allreduce_body,
        out_shape=jax.ShapeDtypeStruct((S, DKV), jnp.float32),
        in_specs=[pl.BlockSpec(memory_space=pltpu.VMEM)],
        out_specs=pl.BlockSpec(memory_space=pltpu.VMEM),
        scratch_shapes=[
            pltpu.VMEM((2, ROWS, DKV), jnp.float32),
            pltpu.SemaphoreType.DMA((2,)),
            pltpu.SemaphoreType.DMA((2,)),
            pltpu.SemaphoreType.REGULAR((2,)),
        ],
        compiler_params=_CompilerParams(collective_id=0),
    )(kvp)


def _allgather_body(mine_ref, out_ref, send_sems, recv_sems, credit_sems):
    d = lax.axis_index("i")
    left = jnp.mod(d - 1, N_DEV)
    right = jnp.mod(d + 1, N_DEV)
    _neighbor_barrier(left, right)

    out_ref[pl.ds(d * ROWS, ROWS), :] = mine_ref[...]

    for t in range(N_DEV - 1):
        slot = t % 2
        if t >= 2:
            _sem_wait(credit_sems.at[slot], 1)
        send_chunk = jnp.mod(d - t, N_DEV)
        recv_chunk = jnp.mod(d - t - 1, N_DEV)
        sr = pl.ds(send_chunk * ROWS, ROWS)
        rr = pl.ds(recv_chunk * ROWS, ROWS)
        send_desc = pltpu.make_async_remote_copy(
            src_ref=out_ref.at[sr, :],
            dst_ref=out_ref.at[sr, :],
            send_sem=send_sems.at[slot],
            recv_sem=recv_sems.at[slot],
            device_id=(right,),
            device_id_type=_DevIdTy.MESH,
        )
        send_desc.start()
        recv_desc = pltpu.make_async_remote_copy(
            src_ref=out_ref.at[rr, :],
            dst_ref=out_ref.at[rr, :],
            send_sem=send_sems.at[slot],
            recv_sem=recv_sems.at[slot],
            device_id=(right,),
            device_id_type=_DevIdTy.MESH,
        )
        send_desc.wait_send()
        recv_desc.wait_recv()
        if t <= N_DEV - 4:
            _sem_signal(
                credit_sems.at[slot], inc=1, device_id=(left,),
                device_id_type=_DevIdTy.MESH,
            )


def _pallas_allgather(mine):
    return pl.pallas_call(
        _allgather_body,
        out_shape=jax.ShapeDtypeStruct((S, DM), jnp.float32),
        in_specs=[pl.BlockSpec(memory_space=pltpu.VMEM)],
        out_specs=pl.BlockSpec(memory_space=pltpu.VMEM),
        scratch_shapes=[
            pltpu.SemaphoreType.DMA((2,)),
            pltpu.SemaphoreType.DMA((2,)),
            pltpu.SemaphoreType.REGULAR((2,)),
        ],
        compiler_params=_CompilerParams(collective_id=1),
    )(mine)


def kernel(x, Wdkv, Wuk, Wuv, Wq, Wqr, Wkr, Wo):
    xm = x[0]
    c = xm @ Wdkv
    kvp = jnp.concatenate([c @ Wuk, c @ Wuv], axis=1)
    kv = _pallas_allreduce(kvp)
    K = kv[:, :H * DH].reshape(S, H, DH)
    V = kv[:, H * DH:].reshape(S, H, DH)

    d = lax.axis_index("i")
    xq = lax.dynamic_slice_in_dim(xm, d * ROWS, ROWS, axis=0)
    Q = (xq @ Wq).reshape(ROWS, H, DH)
    Qr = (xq @ Wqr).reshape(ROWS, H, DR)
    Kr = xm @ Wkr
    scale = (DH + DR) ** -0.5
    scores = (jnp.einsum("shd,thd->hst", Q, K)
              + jnp.einsum("shr,tr->hst", Qr, Kr)) * scale
    P = jax.nn.softmax(scores, axis=-1)
    O = jnp.einsum("hst,thd->shd", P, V).reshape(ROWS, H * DH)
    out_mine = O @ Wo

    out_full = _pallas_allgather(out_mine)
    return out_full[None]


# device time: 430526 ns/iter; 1.5668x vs baseline; 1.5668x over previous
import jax
import jax.numpy as jnp
from jax import lax
from jax.experimental import pallas as pl
from jax.experimental.pallas import tpu as pltpu

N_DEV = 32
S = 1024
DM = 2048
DKV = 4096
HALF = DKV // 2
H, DH, DR = 16, 128, 32
ROWS = S // N_DEV

_sem_signal = getattr(pl, "semaphore_signal", None) or pltpu.semaphore_signal
_sem_wait = getattr(pl, "semaphore_wait", None) or pltpu.semaphore_wait
_DevIdTy = getattr(pl, "DeviceIdType", None) or pltpu.DeviceIdType
_CompilerParams = getattr(pltpu, "CompilerParams", None) or pltpu.TPUCompilerParams


def _rank_of(d):
    z = d // 8
    k = jnp.mod(d, 8)
    y = k // 2
    m = jnp.mod(k, 4)
    x = jnp.where((m == 1) | (m == 2), 1, 0)
    f = 4 * z + jnp.where(jnp.mod(z, 2) == 0, y, 3 - y)
    return jnp.where(x == 0, f, 31 - f)


def _id_at(r):
    r = jnp.mod(r, N_DEV)
    x = jnp.where(r < 16, 0, 1)
    f = jnp.where(r < 16, r, 31 - r)
    z = f // 4
    yy = jnp.mod(f, 4)
    y = jnp.where(jnp.mod(z, 2) == 0, yy, 3 - yy)
    k = 2 * y + jnp.where(jnp.mod(y, 2) == 0, x, 1 - x)
    return 8 * z + k


def _neighbor_barrier(nbr_a, nbr_b):
    barrier = pltpu.get_barrier_semaphore()
    for nbr in (nbr_a, nbr_b):
        _sem_signal(barrier, inc=1, device_id=(nbr,), device_id_type=_DevIdTy.MESH)
    _sem_wait(barrier, 2)


def _allreduce_body(kvp_ref, out_ref, stage_ref, send_sems, recv_sems, credit_sems):
    d = lax.axis_index("i")
    rank = _rank_of(d)
    nxt = _id_at(rank + 1)
    prv = _id_at(rank - 1)
    _neighbor_barrier(nxt, prv)

    out_ref[...] = kvp_ref[...]

    n_rs = N_DEV - 1
    n_steps = 2 * (N_DEV - 1)

    for u in range(n_steps):
        slot = u % 2
        if u >= 2:
            _sem_wait(credit_sems.at[0, slot], 1)
            _sem_wait(credit_sems.at[1, slot], 1)

        descs = []
        for dirn in range(2):
            dst_dev = nxt if dirn == 0 else prv
            col0 = dirn * HALF
            sgn = 1 - 2 * dirn
            if u < n_rs:
                send_chunk = jnp.mod(rank - sgn * u, N_DEV)
                recv_chunk = jnp.mod(rank - sgn * (u + 1), N_DEV)
            else:
                t = u - n_rs
                send_chunk = jnp.mod(rank + sgn * (1 - t), N_DEV)
                recv_chunk = jnp.mod(rank - sgn * t, N_DEV)
            sr = pl.ds(send_chunk * ROWS, ROWS)
            rr = pl.ds(recv_chunk * ROWS, ROWS)
            if u < n_rs:
                rdma = pltpu.make_async_remote_copy(
                    src_ref=out_ref.at[sr, pl.ds(col0, HALF)],
                    dst_ref=stage_ref.at[dirn, slot],
                    send_sem=send_sems.at[dirn, slot],
                    recv_sem=recv_sems.at[dirn, slot],
                    device_id=(dst_dev,),
                    device_id_type=_DevIdTy.MESH,
                )
                rdma.start()
                descs.append((rdma, None, rr, col0))
            else:
                send_desc = pltpu.make_async_remote_copy(
                    src_ref=out_ref.at[sr, pl.ds(col0, HALF)],
                    dst_ref=out_ref.at[sr, pl.ds(col0, HALF)],
                    send_sem=send_sems.at[dirn, slot],
                    recv_sem=recv_sems.at[dirn, slot],
                    device_id=(dst_dev,),
                    device_id_type=_DevIdTy.MESH,
                )
                send_desc.start()
                recv_desc = pltpu.make_async_remote_copy(
                    src_ref=out_ref.at[rr, pl.ds(col0, HALF)],
                    dst_ref=out_ref.at[rr, pl.ds(col0, HALF)],
                    send_sem=send_sems.at[dirn, slot],
                    recv_sem=recv_sems.at[dirn, slot],
                    device_id=(dst_dev,),
                    device_id_type=_DevIdTy.MESH,
                )
                descs.append((send_desc, recv_desc, rr, col0))

        for dirn, (a, b, rr, col0) in enumerate(descs):
            if b is None:
                a.wait()
                out_ref[rr, pl.ds(col0, HALF)] = (
                    out_ref[rr, pl.ds(col0, HALF)] + stage_ref[dirn, slot]
                )
            else:
                a.wait_send()
                b.wait_recv()

        if u <= n_steps - 3:
            _sem_signal(credit_sems.at[0, slot], inc=1, device_id=(prv,),
                        device_id_type=_DevIdTy.MESH)
            _sem_signal(credit_sems.at[1, slot], inc=1, device_id=(nxt,),
                        device_id_type=_DevIdTy.MESH)


def _pallas_allreduce(kvp):
    return pl.pallas_call(
        _allreduce_body,
        out_shape=jax.ShapeDtypeStruct((S, DKV), jnp.float32),
        in_specs=[pl.BlockSpec(memory_space=pltpu.VMEM)],
        out_specs=pl.BlockSpec(memory_space=pltpu.VMEM),
        scratch_shapes=[
            pltpu.VMEM((2, 2, ROWS, HALF), jnp.float32),
            pltpu.SemaphoreType.DMA((2, 2)),
            pltpu.SemaphoreType.DMA((2, 2)),
            pltpu.SemaphoreType.REGULAR((2, 2)),
        ],
        compiler_params=_CompilerParams(collective_id=0),
    )(kvp)


def _allgather_body(mine_ref, out_ref, send_sems, recv_sems, credit_sems):
    d = lax.axis_index("i")
    rank = _rank_of(d)
    nxt = _id_at(rank + 1)
    prv = _id_at(rank - 1)
    _neighbor_barrier(nxt, prv)

    out_ref[pl.ds(d * ROWS, ROWS), :] = mine_ref[...]
    hm = DM // 2

    for t in range(N_DEV - 1):
        slot = t % 2
        if t >= 2:
            _sem_wait(credit_sems.at[0, slot], 1)
            _sem_wait(credit_sems.at[1, slot], 1)
        waits = []
        for dirn in range(2):
            dst_dev = nxt if dirn == 0 else prv
            col0 = dirn * hm
            sgn = 1 - 2 * dirn
            send_chunk = _id_at(rank - sgn * t)
            recv_chunk = _id_at(rank - sgn * (t + 1))
            sr = pl.ds(send_chunk * ROWS, ROWS)
            rr = pl.ds(recv_chunk * ROWS, ROWS)
            send_desc = pltpu.make_async_remote_copy(
                src_ref=out_ref.at[sr, pl.ds(col0, hm)],
                dst_ref=out_ref.at[sr, pl.ds(col0, hm)],
                send_sem=send_sems.at[dirn, slot],
                recv_sem=recv_sems.at[dirn, slot],
                device_id=(dst_dev,),
                device_id_type=_DevIdTy.MESH,
            )
            send_desc.start()
            recv_desc = pltpu.make_async_remote_copy(
                src_ref=out_ref.at[rr, pl.ds(col0, hm)],
                dst_ref=out_ref.at[rr, pl.ds(col0, hm)],
                send_sem=send_sems.at[dirn, slot],
                recv_sem=recv_sems.at[dirn, slot],
                device_id=(dst_dev,),
                device_id_type=_DevIdTy.MESH,
            )
            waits.append((send_desc, recv_desc))
        for send_desc, recv_desc in waits:
            send_desc.wait_send()
            recv_desc.wait_recv()
        if t <= N_DEV - 4:
            _sem_signal(credit_sems.at[0, slot], inc=1, device_id=(prv,),
                        device_id_type=_DevIdTy.MESH)
            _sem_signal(credit_sems.at[1, slot], inc=1, device_id=(nxt,),
                        device_id_type=_DevIdTy.MESH)


def _pallas_allgather(mine):
    return pl.pallas_call(
        _allgather_body,
        out_shape=jax.ShapeDtypeStruct((S, DM), jnp.float32),
        in_specs=[pl.BlockSpec(memory_space=pltpu.VMEM)],
        out_specs=pl.BlockSpec(memory_space=pltpu.VMEM),
        scratch_shapes=[
            pltpu.SemaphoreType.DMA((2, 2)),
            pltpu.SemaphoreType.DMA((2, 2)),
            pltpu.SemaphoreType.REGULAR((2, 2)),
        ],
        compiler_params=_CompilerParams(collective_id=1),
    )(mine)


def kernel(x, Wdkv, Wuk, Wuv, Wq, Wqr, Wkr, Wo):
    xm = x[0]
    c = xm @ Wdkv
    kvp = jnp.concatenate([c @ Wuk, c @ Wuv], axis=1)
    kv = _pallas_allreduce(kvp)
    K = kv[:, :H * DH].reshape(S, H, DH)
    V = kv[:, H * DH:].reshape(S, H, DH)

    d = lax.axis_index("i")
    xq = lax.dynamic_slice_in_dim(xm, d * ROWS, ROWS, axis=0)
    Q = (xq @ Wq).reshape(ROWS, H, DH)
    Qr = (xq @ Wqr).reshape(ROWS, H, DR)
    Kr = xm @ Wkr
    scale = (DH + DR) ** -0.5
    scores = (jnp.einsum("shd,thd->hst", Q, K)
              + jnp.einsum("shr,tr->hst", Qr, Kr)) * scale
    P = jax.nn.softmax(scores, axis=-1)
    O = jnp.einsum("hst,thd->shd", P, V).reshape(ROWS, H * DH)
    out_mine = O @ Wo

    out_full = _pallas_allgather(out_mine)
    return out_full[None]
